# baseline (speedup 1.0000x reference)
import functools

import jax
import jax.numpy as jnp
from jax import lax
from jax.experimental import pallas as pl
from jax.experimental.pallas import tpu as pltpu
from jax.experimental.pallas import tpu_sc as plsc

_N = 100000
_D = 256
_G = 128
_NCORES = 2
_NSUB = 16
_NW = _NCORES * _NSUB
_RPT = _N // _NW
_CHUNK = 64
_CSZ = _CHUNK * _D
_NFULL = _RPT // _CHUNK
_TAIL = _RPT - _NFULL * _CHUNK
_TFULL = _TAIL // 16
_TREM = _TAIL - _TFULL * 16
_RPW = _G // _NSUB
_NVR = _D // 16
_AW = _D + 16
_ASZ = _G * _AW
_WSZ = _RPW * _AW


def _sc_segment_sum(nodef, ids2d, strag, sids):
    mesh = plsc.VectorSubcoreMesh(core_axis_name="c", subcore_axis_name="s")

    @functools.partial(
        pl.kernel,
        mesh=mesh,
        out_type=jax.ShapeDtypeStruct((_NCORES, _ASZ), jnp.float32),
        scratch_types=[
            pltpu.VMEM((_RPT,), jnp.int32),
            pltpu.VMEM((_CSZ,), jnp.float32),
            pltpu.VMEM((_CSZ,), jnp.float32),
            pltpu.VMEM((_ASZ,), jnp.float32),
            pltpu.VMEM((_WSZ,), jnp.float32),
            pltpu.VMEM((_WSZ,), jnp.float32),
            pltpu.VMEM((8 * _D,), jnp.float32),
            pltpu.VMEM((16,), jnp.int32),
            pltpu.VMEM_SHARED((_NSUB, _ASZ), jnp.float32),
            pltpu.SemaphoreType.DMA,
            pltpu.SemaphoreType.DMA,
        ],
    )
    def k(node_hbm, ids_hbm, strag_hbm, sids_hbm, out_acc,
          idx_v, buf_a, buf_b, acc_v, stage_v, res_v, buf_t, idxt_v,
          sh_acc, sem_a, sem_b):
        c = lax.axis_index("c")
        s = lax.axis_index("s")
        wid = c * _NSUB + s

        def _zrow(i, carry):
            acc_v[pl.ds(i * 16, 16)] = jnp.zeros((16,), jnp.float32)
            return carry
        lax.fori_loop(0, _ASZ // 16, _zrow, 0)

        pltpu.sync_copy(ids_hbm.at[wid], idx_v)
        pltpu.async_copy(node_hbm.at[wid, pl.ds(0, _CSZ)], buf_a, sem_a)
        pltpu.async_copy(node_hbm.at[wid, pl.ds(_CSZ, _CSZ)], buf_b, sem_b)

        ones = jnp.ones((16,), jnp.float32)
        zero16 = jnp.zeros((16,), jnp.float32)

        def _rows(buf, boff, sidv, lanes, carry):
            prev = carry[0]
            regs = list(carry[1:])
            for kk, lane in lanes:
                sid = sidv[lane]
                keep = lax.broadcast(
                    lax.convert_element_type(sid == prev, jnp.float32), (16,))
                base = boff + kk * _D
                abase = sid * _AW
                new = []
                for j in range(_NVR):
                    r = buf[pl.ds(base + j * 16, 16)]
                    v = regs[j] * keep + r
                    acc_v[pl.ds(abase + j * 16, 16)] = v
                    new.append(v)
                vc = regs[_NVR] * keep + ones
                acc_v[pl.ds(abase + _D, 16)] = vc
                regs = new + [vc]
                prev = sid
            return (prev, *regs)

        def _group(ioff, boff, buf, carry):
            sidv = idx_v[pl.ds(ioff, 16)]
            return _rows(buf, boff, sidv, [(kk, kk) for kk in range(16)], carry)

        def _chunk(t, buf, carry):
            def _g(g, carry):
                return _group(t * _CHUNK + g * 16, g * 16 * _D, buf, carry)
            return lax.fori_loop(0, _CHUNK // 16, _g, carry)

        carry = (jnp.int32(-1),) + tuple(
            jnp.zeros((16,), jnp.float32) for _ in range(_NVR + 1))

        def _pair(m, carry):
            t0 = 2 * m
            pltpu.make_async_copy(
                node_hbm.at[wid, pl.ds(t0 * _CSZ, _CSZ)], buf_a, sem_a
            ).wait()
            carry = _chunk(t0, buf_a, carry)

            @pl.when(m < _NFULL // 2 - 1)
            def _():
                pltpu.async_copy(
                    node_hbm.at[wid, pl.ds((t0 + 2) * _CSZ, _CSZ)],
                    buf_a, sem_a)
            pltpu.make_async_copy(
                node_hbm.at[wid, pl.ds((t0 + 1) * _CSZ, _CSZ)], buf_b, sem_b
            ).wait()
            carry = _chunk(t0 + 1, buf_b, carry)

            @pl.when(m < _NFULL // 2 - 1)
            def _():
                pltpu.async_copy(
                    node_hbm.at[wid, pl.ds((t0 + 3) * _CSZ, _CSZ)],
                    buf_b, sem_b)
            return carry

        carry = lax.fori_loop(0, _NFULL // 2, _pair, carry)

        pltpu.sync_copy(
            node_hbm.at[wid, pl.ds(_NFULL * _CSZ, _TFULL * 16 * _D)],
            buf_a.at[pl.ds(0, _TFULL * 16 * _D)])

        def _tg(g, carry):
            return _group(_NFULL * _CHUNK + g * 16, g * 16 * _D, buf_a, carry)
        carry = lax.fori_loop(0, _TFULL, _tg, carry)

        pltpu.sync_copy(strag_hbm.at[wid], buf_t)
        pltpu.sync_copy(sids_hbm.at[wid], idxt_v)
        sidv_t = idxt_v[...]
        carry = _rows(buf_t, 0, sidv_t, [(kk, kk) for kk in range(_TREM)],
                      carry)

        pltpu.sync_copy(acc_v, sh_acc.at[s])
        plsc.subcore_barrier()

        def _zres(i, carry):
            res_v[pl.ds(i * 16, 16)] = jnp.zeros((16,), jnp.float32)
            return carry
        lax.fori_loop(0, _WSZ // 16, _zres, 0)

        def _red(p, carry):
            pltpu.sync_copy(sh_acc.at[p, pl.ds(s * _WSZ, _WSZ)], stage_v)
            def _radd(i, carry2):
                sl = pl.ds(i * 16, 16)
                res_v[sl] = res_v[sl] + stage_v[sl]
                return carry2
            lax.fori_loop(0, _WSZ // 16, _radd, 0)
            return carry
        lax.fori_loop(0, _NSUB, _red, 0)
        pltpu.sync_copy(res_v, out_acc.at[c, pl.ds(s * _WSZ, _WSZ)])

    return k(nodef, ids2d, strag, sids)


def _dense_body(pacc_ref, w1_ref, b1_ref, w2_ref, b2_ref, logp_ref, gs_ref):
    pacc = pacc_ref[...]
    part = pacc[0] + pacc[1]
    seg = part[:, :_D]
    cnt = part[:, _D:_D + 1]
    gs = lax.dot(seg, w1_ref[...], precision=lax.Precision.HIGHEST)
    gs = gs + cnt * b1_ref[...]
    logits = lax.dot(gs, w2_ref[...], precision=lax.Precision.HIGHEST)
    logits = logits + b2_ref[...]
    m = jnp.max(logits, axis=1, keepdims=True)
    lse = m + jnp.log(jnp.sum(jnp.exp(logits - m), axis=1, keepdims=True))
    logp_ref[...] = logits - lse
    gs_ref[...] = gs


def kernel(node_features, batch_segments, num_graphs, W1, b1, W2, b2):
    del num_graphs
    d_hid = W1.shape[1]
    n_cls = W2.shape[1]
    node3d = node_features.reshape(_NW, _RPT, _D)
    ids2d = batch_segments.astype(jnp.int32).reshape(_NW, _RPT)
    strag = jnp.pad(node3d[:, _RPT - _TREM:, :],
                    ((0, 0), (0, 8 - _TREM), (0, 0))).reshape(_NW, 8 * _D)
    sids = jnp.pad(ids2d[:, _RPT - _TREM:], ((0, 0), (0, 16 - _TREM)))

    pacc = _sc_segment_sum(node3d.reshape(_NW, _RPT * _D), ids2d, strag, sids)
    pacc = pacc.reshape(_NCORES, _G, _AW)

    logp, gs = pl.pallas_call(
        _dense_body,
        out_shape=(
            jax.ShapeDtypeStruct((_G, n_cls), jnp.float32),
            jax.ShapeDtypeStruct((_G, d_hid), jnp.float32),
        ),
    )(pacc, W1, b1.reshape(1, d_hid), W2, b2.reshape(1, n_cls))
    return (logp, gs)


# speedup vs baseline: 3.9261x; 3.9261x over previous
import functools

import jax
import jax.numpy as jnp
from jax import lax
from jax.experimental import pallas as pl
from jax.experimental.pallas import tpu as pltpu
from jax.experimental.pallas import tpu_sc as plsc

_N = 100000
_D = 256
_G = 128
_NCORES = 2
_NSUB = 16
_NW = _NCORES * _NSUB
_RPT = _N // _NW
_CHUNK = 64
_CSZ = _CHUNK * _D
_NFULL = _RPT // _CHUNK
_TAIL = _RPT - _NFULL * _CHUNK
_TFULL = _TAIL // 16
_TREM = _TAIL - _TFULL * 16
_RPW = _G // _NSUB
_NVR = _D // 16
_AW = _D + 16
_ASZ = _G * _AW
_WSZ = _RPW * _AW


def _sc_segment_sum(nodef, ids2d, strag, sids):
    mesh = plsc.VectorSubcoreMesh(core_axis_name="c", subcore_axis_name="s")

    @functools.partial(
        pl.kernel,
        mesh=mesh,
        out_type=jax.ShapeDtypeStruct((_NCORES, _ASZ), jnp.float32),
        scratch_types=[
            pltpu.VMEM((_RPT,), jnp.int32),
            pltpu.VMEM((_CSZ,), jnp.float32),
            pltpu.VMEM((_CSZ,), jnp.float32),
            pltpu.VMEM((_ASZ,), jnp.float32),
            pltpu.VMEM((_WSZ,), jnp.float32),
            pltpu.VMEM((_WSZ,), jnp.float32),
            pltpu.VMEM((8 * _D,), jnp.float32),
            pltpu.VMEM((16,), jnp.int32),
            pltpu.VMEM_SHARED((_NSUB, _ASZ), jnp.float32),
            pltpu.SemaphoreType.DMA,
            pltpu.SemaphoreType.DMA,
        ],
    )
    def k(node_hbm, ids_hbm, strag_hbm, sids_hbm, out_acc,
          idx_v, buf_a, buf_b, acc_v, stage_v, res_v, buf_t, idxt_v,
          sh_acc, sem_a, sem_b):
        c = lax.axis_index("c")
        s = lax.axis_index("s")
        wid = c * _NSUB + s
        woff = pl.multiple_of(wid * (_RPT * _D), _D)

        def _zrow(i, carry):
            acc_v[pl.ds(i * 16, 16)] = jnp.zeros((16,), jnp.float32)
            return carry
        lax.fori_loop(0, _ASZ // 16, _zrow, 0)

        pltpu.sync_copy(ids_hbm.at[wid], idx_v)
        pltpu.async_copy(node_hbm.at[pl.ds(woff, _CSZ)], buf_a, sem_a)
        pltpu.async_copy(node_hbm.at[pl.ds(woff + _CSZ, _CSZ)], buf_b, sem_b)

        ones = jnp.ones((16,), jnp.float32)
        zero16 = jnp.zeros((16,), jnp.float32)

        def _rows(buf, boff, sidv, lanes, carry):
            prev = carry[0]
            regs = list(carry[1:])
            for kk, lane in lanes:
                sid = sidv[lane]
                keep = lax.broadcast(
                    lax.convert_element_type(sid == prev, jnp.float32), (16,))
                base = boff + kk * _D
                abase = sid * _AW
                new = []
                for j in range(_NVR):
                    r = buf[pl.ds(base + j * 16, 16)]
                    v = regs[j] * keep + r
                    acc_v[pl.ds(abase + j * 16, 16)] = v
                    new.append(v)
                vc = regs[_NVR] * keep + ones
                acc_v[pl.ds(abase + _D, 16)] = vc
                regs = new + [vc]
                prev = sid
            return (prev, *regs)

        def _group(ioff, boff, buf, carry):
            sidv = idx_v[pl.ds(ioff, 16)]
            return _rows(buf, boff, sidv, [(kk, kk) for kk in range(16)], carry)

        def _chunk(t, buf, carry):
            def _g(g, carry):
                return _group(t * _CHUNK + g * 16, g * 16 * _D, buf, carry)
            return lax.fori_loop(0, _CHUNK // 16, _g, carry)

        carry = (jnp.int32(-1),) + tuple(
            jnp.zeros((16,), jnp.float32) for _ in range(_NVR + 1))

        def _pair(m, carry):
            t0 = 2 * m
            pltpu.make_async_copy(
                node_hbm.at[pl.ds(woff + t0 * _CSZ, _CSZ)], buf_a, sem_a
            ).wait()
            carry = _chunk(t0, buf_a, carry)

            @pl.when(m < _NFULL // 2 - 1)
            def _():
                pltpu.async_copy(
                    node_hbm.at[pl.ds(woff + (t0 + 2) * _CSZ, _CSZ)],
                    buf_a, sem_a)
            pltpu.make_async_copy(
                node_hbm.at[pl.ds(woff + (t0 + 1) * _CSZ, _CSZ)], buf_b, sem_b
            ).wait()
            carry = _chunk(t0 + 1, buf_b, carry)

            @pl.when(m < _NFULL // 2 - 1)
            def _():
                pltpu.async_copy(
                    node_hbm.at[pl.ds(woff + (t0 + 3) * _CSZ, _CSZ)],
                    buf_b, sem_b)
            return carry

        carry = lax.fori_loop(0, _NFULL // 2, _pair, carry)

        pltpu.sync_copy(
            node_hbm.at[pl.ds(woff + _NFULL * _CSZ, _TFULL * 16 * _D)],
            buf_a.at[pl.ds(0, _TFULL * 16 * _D)])

        def _tg(g, carry):
            return _group(_NFULL * _CHUNK + g * 16, g * 16 * _D, buf_a, carry)
        carry = lax.fori_loop(0, _TFULL, _tg, carry)

        pltpu.sync_copy(strag_hbm.at[wid], buf_t)
        pltpu.sync_copy(sids_hbm.at[wid], idxt_v)
        sidv_t = idxt_v[...]
        carry = _rows(buf_t, 0, sidv_t, [(kk, kk) for kk in range(_TREM)],
                      carry)

        pltpu.sync_copy(acc_v, sh_acc.at[s])
        plsc.subcore_barrier()

        def _zres(i, carry):
            res_v[pl.ds(i * 16, 16)] = jnp.zeros((16,), jnp.float32)
            return carry
        lax.fori_loop(0, _WSZ // 16, _zres, 0)

        def _red(p, carry):
            pltpu.sync_copy(sh_acc.at[p, pl.ds(s * _WSZ, _WSZ)], stage_v)
            def _radd(i, carry2):
                sl = pl.ds(i * 16, 16)
                res_v[sl] = res_v[sl] + stage_v[sl]
                return carry2
            lax.fori_loop(0, _WSZ // 16, _radd, 0)
            return carry
        lax.fori_loop(0, _NSUB, _red, 0)
        pltpu.sync_copy(res_v, out_acc.at[c, pl.ds(s * _WSZ, _WSZ)])

    return k(nodef, ids2d, strag, sids)


def _dense_body(pacc_ref, w1_ref, b1_ref, w2_ref, b2_ref, logp_ref, gs_ref):
    pacc = pacc_ref[...]
    part = pacc[0] + pacc[1]
    seg = part[:, :_D]
    cnt = part[:, _D:_D + 1]
    gs = lax.dot(seg, w1_ref[...], precision=lax.Precision.HIGHEST)
    gs = gs + cnt * b1_ref[...]
    logits = lax.dot(gs, w2_ref[...], precision=lax.Precision.HIGHEST)
    logits = logits + b2_ref[...]
    m = jnp.max(logits, axis=1, keepdims=True)
    lse = m + jnp.log(jnp.sum(jnp.exp(logits - m), axis=1, keepdims=True))
    logp_ref[...] = logits - lse
    gs_ref[...] = gs


def kernel(node_features, batch_segments, num_graphs, W1, b1, W2, b2):
    del num_graphs
    d_hid = W1.shape[1]
    n_cls = W2.shape[1]
    nodef = node_features.reshape(_N * _D)
    ids2d = batch_segments.astype(jnp.int32).reshape(_NW, _RPT)
    rid = (jnp.arange(_NW) * _RPT + (_RPT - _TREM))[:, None] \
        + jnp.arange(_TREM)[None, :]
    fid = rid[:, :, None] * _D + jnp.arange(_D)[None, None, :]
    strag = jnp.pad(nodef[fid], ((0, 0), (0, 8 - _TREM), (0, 0)))
    strag = strag.reshape(_NW, 8 * _D)
    sids = jnp.pad(ids2d[:, _RPT - _TREM:], ((0, 0), (0, 16 - _TREM)))

    pacc = _sc_segment_sum(nodef, ids2d, strag, sids)
    pacc = pacc.reshape(_NCORES, _G, _AW)

    logp, gs = pl.pallas_call(
        _dense_body,
        out_shape=(
            jax.ShapeDtypeStruct((_G, n_cls), jnp.float32),
            jax.ShapeDtypeStruct((_G, d_hid), jnp.float32),
        ),
    )(pacc, W1, b1.reshape(1, d_hid), W2, b2.reshape(1, n_cls))
    return (logp, gs)


# speedup vs baseline: 9.2843x; 2.3648x over previous
import functools

import jax
import jax.numpy as jnp
from jax import lax
from jax.experimental import pallas as pl
from jax.experimental.pallas import tpu as pltpu
from jax.experimental.pallas import tpu_sc as plsc

_N = 100000
_D = 256
_G = 128
_NCORES = 2
_NSUB = 16
_NW = _NCORES * _NSUB
_RPT = _N // _NW
_CHUNK = 64
_CSZ = _CHUNK * _D
_NFULL = _RPT // _CHUNK
_TAIL = _RPT - _NFULL * _CHUNK
_TFULL = _TAIL // 16
_TREM = _TAIL - _TFULL * 16
_RPW = _G // _NSUB
_NVR = _D // 16
_AW = _D + 16
_ASZ = _G * _AW
_WSZ = _RPW * _AW


def _sc_segment_sum(nodef, ids2d, strag, sids):
    mesh = plsc.VectorSubcoreMesh(core_axis_name="c", subcore_axis_name="s")

    @functools.partial(
        pl.kernel,
        mesh=mesh,
        out_type=jax.ShapeDtypeStruct((_NCORES, _ASZ), jnp.float32),
        scratch_types=[
            pltpu.VMEM((_RPT,), jnp.int32),
            pltpu.VMEM((_CSZ,), jnp.float32),
            pltpu.VMEM((_CSZ,), jnp.float32),
            pltpu.VMEM((_ASZ,), jnp.float32),
            pltpu.VMEM((_WSZ,), jnp.float32),
            pltpu.VMEM((_WSZ,), jnp.float32),
            pltpu.VMEM((8 * _D,), jnp.float32),
            pltpu.VMEM((16,), jnp.int32),
            pltpu.VMEM_SHARED((_NSUB, _ASZ), jnp.float32),
            pltpu.SemaphoreType.DMA,
            pltpu.SemaphoreType.DMA,
        ],
    )
    def k(node_hbm, ids_hbm, strag_hbm, sids_hbm, out_acc,
          idx_v, buf_a, buf_b, acc_v, stage_v, res_v, buf_t, idxt_v,
          sh_acc, sem_a, sem_b):
        c = lax.axis_index("c")
        s = lax.axis_index("s")
        wid = c * _NSUB + s
        woff = pl.multiple_of(wid * (_RPT * _D), _D)

        def _zrow(i, carry):
            acc_v[pl.ds(i * 16, 16)] = jnp.zeros((16,), jnp.float32)
            return carry
        lax.fori_loop(0, _ASZ // 16, _zrow, 0)

        pltpu.sync_copy(ids_hbm.at[wid], idx_v)
        pltpu.async_copy(node_hbm.at[pl.ds(woff, _CSZ)], buf_a, sem_a)
        pltpu.async_copy(node_hbm.at[pl.ds(woff + _CSZ, _CSZ)], buf_b, sem_b)

        ones = jnp.ones((16,), jnp.float32)
        zero16 = jnp.zeros((16,), jnp.float32)

        def _rows(buf, boff, sidv, lanes, carry):
            prev = carry[0]
            regs = list(carry[1:])
            for kk, lane in lanes:
                sid = sidv[lane]
                base = boff + kk * _D

                def _flush(prev=prev, regs=regs):
                    b = prev * _AW
                    for j in range(_NVR + 1):
                        acc_v[pl.ds(b + j * 16, 16)] = regs[j]
                    return (zero16,) * (_NVR + 1)

                def _keep(regs=regs):
                    return tuple(regs)

                regs = list(lax.cond(sid != prev, _flush, _keep))
                for j in range(_NVR):
                    regs[j] = regs[j] + buf[pl.ds(base + j * 16, 16)]
                regs[_NVR] = regs[_NVR] + ones
                prev = sid
            return (prev, *regs)

        def _group(ioff, boff, buf, carry):
            sidv = idx_v[pl.ds(ioff, 16)]
            return _rows(buf, boff, sidv, [(kk, kk) for kk in range(16)], carry)

        def _chunk(t, buf, carry):
            def _g(g, carry):
                return _group(t * _CHUNK + g * 16, g * 16 * _D, buf, carry)
            return lax.fori_loop(0, _CHUNK // 16, _g, carry)

        sid0 = idx_v[pl.ds(0, 16)][0]
        carry = (sid0,) + tuple(
            jnp.zeros((16,), jnp.float32) for _ in range(_NVR + 1))

        def _pair(m, carry):
            t0 = 2 * m
            pltpu.make_async_copy(
                node_hbm.at[pl.ds(woff + t0 * _CSZ, _CSZ)], buf_a, sem_a
            ).wait()
            carry = _chunk(t0, buf_a, carry)

            @pl.when(m < _NFULL // 2 - 1)
            def _():
                pltpu.async_copy(
                    node_hbm.at[pl.ds(woff + (t0 + 2) * _CSZ, _CSZ)],
                    buf_a, sem_a)
            pltpu.make_async_copy(
                node_hbm.at[pl.ds(woff + (t0 + 1) * _CSZ, _CSZ)], buf_b, sem_b
            ).wait()
            carry = _chunk(t0 + 1, buf_b, carry)

            @pl.when(m < _NFULL // 2 - 1)
            def _():
                pltpu.async_copy(
                    node_hbm.at[pl.ds(woff + (t0 + 3) * _CSZ, _CSZ)],
                    buf_b, sem_b)
            return carry

        carry = lax.fori_loop(0, _NFULL // 2, _pair, carry)

        pltpu.sync_copy(
            node_hbm.at[pl.ds(woff + _NFULL * _CSZ, _TFULL * 16 * _D)],
            buf_a.at[pl.ds(0, _TFULL * 16 * _D)])

        def _tg(g, carry):
            return _group(_NFULL * _CHUNK + g * 16, g * 16 * _D, buf_a, carry)
        carry = lax.fori_loop(0, _TFULL, _tg, carry)

        pltpu.sync_copy(strag_hbm.at[wid], buf_t)
        pltpu.sync_copy(sids_hbm.at[wid], idxt_v)
        sidv_t = idxt_v[...]
        carry = _rows(buf_t, 0, sidv_t, [(kk, kk) for kk in range(_TREM)],
                      carry)

        fprev = carry[0]
        fregs = carry[1:]
        fb = fprev * _AW
        for j in range(_NVR + 1):
            acc_v[pl.ds(fb + j * 16, 16)] = fregs[j]

        pltpu.sync_copy(acc_v, sh_acc.at[s])
        plsc.subcore_barrier()

        def _zres(i, carry):
            res_v[pl.ds(i * 16, 16)] = jnp.zeros((16,), jnp.float32)
            return carry
        lax.fori_loop(0, _WSZ // 16, _zres, 0)

        def _red(p, carry):
            pltpu.sync_copy(sh_acc.at[p, pl.ds(s * _WSZ, _WSZ)], stage_v)
            def _radd(i, carry2):
                sl = pl.ds(i * 16, 16)
                res_v[sl] = res_v[sl] + stage_v[sl]
                return carry2
            lax.fori_loop(0, _WSZ // 16, _radd, 0)
            return carry
        lax.fori_loop(0, _NSUB, _red, 0)
        pltpu.sync_copy(res_v, out_acc.at[c, pl.ds(s * _WSZ, _WSZ)])

    return k(nodef, ids2d, strag, sids)


def _dense_body(pacc_ref, w1_ref, b1_ref, w2_ref, b2_ref, logp_ref, gs_ref):
    pacc = pacc_ref[...]
    part = pacc[0] + pacc[1]
    seg = part[:, :_D]
    cnt = part[:, _D:_D + 1]
    gs = lax.dot(seg, w1_ref[...], precision=lax.Precision.HIGHEST)
    gs = gs + cnt * b1_ref[...]
    logits = lax.dot(gs, w2_ref[...], precision=lax.Precision.HIGHEST)
    logits = logits + b2_ref[...]
    m = jnp.max(logits, axis=1, keepdims=True)
    lse = m + jnp.log(jnp.sum(jnp.exp(logits - m), axis=1, keepdims=True))
    logp_ref[...] = logits - lse
    gs_ref[...] = gs


def kernel(node_features, batch_segments, num_graphs, W1, b1, W2, b2):
    del num_graphs
    d_hid = W1.shape[1]
    n_cls = W2.shape[1]
    nodef = node_features.reshape(_N * _D)
    ids2d = batch_segments.astype(jnp.int32).reshape(_NW, _RPT)
    rid = (jnp.arange(_NW) * _RPT + (_RPT - _TREM))[:, None] \
        + jnp.arange(_TREM)[None, :]
    fid = rid[:, :, None] * _D + jnp.arange(_D)[None, None, :]
    strag = jnp.pad(nodef[fid], ((0, 0), (0, 8 - _TREM), (0, 0)))
    strag = strag.reshape(_NW, 8 * _D)
    sids = jnp.pad(ids2d[:, _RPT - _TREM:], ((0, 0), (0, 16 - _TREM)))

    pacc = _sc_segment_sum(nodef, ids2d, strag, sids)
    pacc = pacc.reshape(_NCORES, _G, _AW)

    logp, gs = pl.pallas_call(
        _dense_body,
        out_shape=(
            jax.ShapeDtypeStruct((_G, n_cls), jnp.float32),
            jax.ShapeDtypeStruct((_G, d_hid), jnp.float32),
        ),
    )(pacc, W1, b1.reshape(1, d_hid), W2, b2.reshape(1, n_cls))
    return (logp, gs)
